# baseline (speedup 1.0000x reference)
import functools

import jax
import jax.numpy as jnp
from jax import lax
from jax.experimental import pallas as pl
from jax.experimental.pallas import tpu as pltpu
from jax.experimental.pallas import tpu_sc as plsc

N = 10000
E = 320000
F = 128
NP = 10240
ROWS_PT = NP // 16
G = 64
CH = 16
N0G = 240
N1G = 80
TOT_G = 16 * (N0G + N1G)
EP = TOT_G * G
BLK = 1280
GRID = NP // BLK

_mesh = plsc.VectorSubcoreMesh(core_axis_name="c", subcore_axis_name="s")


@functools.partial(
    pl.kernel,
    out_type=jax.ShapeDtypeStruct((2, NP), jnp.float32),
    mesh=_mesh,
    scratch_types=[
        pltpu.VMEM((8, G), jnp.int32),
        pltpu.VMEM((G,), jnp.float32),
        pltpu.VMEM_SHARED((NP,), jnp.float32),
    ],
)
def _deg_kernel(dst_hbm, zeros1_hbm, degp_hbm, idx_v, ones_v, acc_sh):
    cid = lax.axis_index("c")
    tid = lax.axis_index("s")
    ng = TOT_G // 32
    base = (cid * 16 + tid) * ng
    pltpu.sync_copy(zeros1_hbm, acc_sh.at[pl.ds(tid * ROWS_PT, ROWS_PT)])
    for i in range(G // 16):
        ones_v[pl.ds(i * 16, 16)] = jnp.ones((16,), jnp.float32)
    plsc.subcore_barrier()

    def body(c, carry):
        pltpu.sync_copy(dst_hbm.at[pl.ds(base + c * 8, 8)], idx_v)

        def inner(k, carry2):
            pltpu.sync_copy(ones_v, acc_sh.at[idx_v.at[k]], add=True)
            return carry2

        return lax.fori_loop(0, 8, inner, carry)

    lax.fori_loop(0, ng // 8, body, 0)
    plsc.subcore_barrier()
    pltpu.sync_copy(acc_sh.at[pl.ds(tid * ROWS_PT, ROWS_PT)],
                    degp_hbm.at[cid, pl.ds(tid * ROWS_PT, ROWS_PT)])


@functools.partial(
    pl.kernel,
    out_type=jax.ShapeDtypeStruct((2, NP, F), jnp.float32),
    mesh=_mesh,
    scratch_types=[
        pltpu.VMEM((2, CH, G), jnp.int32),
        pltpu.VMEM((2, CH, G), jnp.int32),
        pltpu.VMEM((G, F), jnp.float32),
        pltpu.VMEM((G, F), jnp.float32),
        pltpu.VMEM((64, F), jnp.float32),
        pltpu.VMEM_SHARED((NP, F), jnp.float32),
        pltpu.SemaphoreType.DMA,
        pltpu.SemaphoreType.DMA,
        pltpu.SemaphoreType.DMA,
        pltpu.SemaphoreType.DMA,
    ],
)
def _scatter_kernel(hn_hbm, src_hbm, dst_hbm, zrows_hbm, out_hbm,
                    isrc, idst, buf_a, buf_b, zblk, acc_sh,
                    sem_a, sem_b, sem_si, sem_di):
    cid = lax.axis_index("c")
    tid = lax.axis_index("s")
    ng = jnp.where(cid == 0, N0G, N1G)
    nch = ng // CH
    base = jnp.where(cid == 0, tid * N0G, 16 * N0G + tid * N1G)
    pltpu.sync_copy(zrows_hbm, zblk)
    for k in range(ROWS_PT // 64):
        pltpu.sync_copy(zblk, acc_sh.at[pl.ds(tid * ROWS_PT + k * 64, 64)])
    pltpu.sync_copy(src_hbm.at[pl.ds(base, CH)], isrc.at[0])
    pltpu.sync_copy(dst_hbm.at[pl.ds(base, CH)], idst.at[0])
    plsc.subcore_barrier()

    pltpu.async_copy(src_hbm.at[pl.ds(base + CH, CH)], isrc.at[1], sem_si)
    pltpu.async_copy(dst_hbm.at[pl.ds(base + CH, CH)], idst.at[1], sem_di)
    pltpu.async_copy(hn_hbm.at[isrc.at[0, 0]], buf_a, sem_a)
    pltpu.async_copy(hn_hbm.at[isrc.at[0, 1]], buf_b, sem_b)

    def body(g, carry):
        j = 2 * g
        c = j // CH
        r = j - c * CH

        @pl.when((r == CH - 2) & (c + 1 < nch))
        def _():
            pltpu.make_async_copy(src_hbm.at[pl.ds(base, CH)],
                                  isrc.at[0], sem_si).wait()
            pltpu.make_async_copy(dst_hbm.at[pl.ds(base, CH)],
                                  idst.at[0], sem_di).wait()

        def slotrow(jj):
            cc = jj // CH
            return lax.rem(cc, 2), jj - cc * CH

        s0, r0 = lax.rem(c, 2), r
        pltpu.make_async_copy(hn_hbm.at[isrc.at[s0, r0]], buf_a, sem_a).wait()
        pltpu.sync_copy(buf_a, acc_sh.at[idst.at[s0, r0]], add=True)

        @pl.when(j + 2 < ng)
        def _():
            s2, r2 = slotrow(j + 2)
            pltpu.async_copy(hn_hbm.at[isrc.at[s2, r2]], buf_a, sem_a)

        s1, r1 = slotrow(j + 1)
        pltpu.make_async_copy(hn_hbm.at[isrc.at[s1, r1]], buf_b, sem_b).wait()
        pltpu.sync_copy(buf_b, acc_sh.at[idst.at[s1, r1]], add=True)

        @pl.when(j + 3 < ng)
        def _():
            s3, r3 = slotrow(j + 3)
            pltpu.async_copy(hn_hbm.at[isrc.at[s3, r3]], buf_b, sem_b)

        @pl.when((r == CH - 2) & (c + 2 < nch))
        def _():
            slot = lax.rem(c, 2)
            off = base + (c + 2) * CH
            pltpu.async_copy(src_hbm.at[pl.ds(off, CH)], isrc.at[slot], sem_si)
            pltpu.async_copy(dst_hbm.at[pl.ds(off, CH)], idst.at[slot], sem_di)

        return carry

    lax.fori_loop(0, ng // 2, body, 0)
    plsc.subcore_barrier()
    pltpu.sync_copy(acc_sh.at[pl.ds(tid * ROWS_PT, ROWS_PT)],
                    out_hbm.at[cid, pl.ds(tid * ROWS_PT, ROWS_PT)])


def _mm_t(x, w):
    return lax.dot_general(x, w, (((1,), (1,)), ((), ())),
                           preferred_element_type=jnp.float32)


def _prep_body(xp_ref, degp_ref, wproj_ref, bproj_ref, w0_ref,
               h_ref, hn0_ref, dinv_ref):
    d = degp_ref[0] + degp_ref[1] + 1.0
    dv = lax.rsqrt(d)
    h = _mm_t(xp_ref[...], wproj_ref[...]) + bproj_ref[...]
    h_ref[...] = h
    hn0_ref[...] = dv * _mm_t(h, w0_ref[...])
    dinv_ref[...] = dv


def _layer_body(sp_ref, hn_ref, h_ref, dinv_ref, b_ref, wn_ref,
                hout_ref, hnout_ref):
    dv = dinv_ref[...]
    s = sp_ref[0] + sp_ref[1] + hn_ref[...]
    z = dv * s + b_ref[...] + h_ref[...]
    h_new = jnp.where(z >= 0, z, 0.01 * z)
    hout_ref[...] = h_new
    hnout_ref[...] = dv * _mm_t(h_new, wn_ref[...])


def _final_body(sp_ref, hn_ref, h_ref, dinv_ref, b2_ref, xp_ref,
                wlin_ref, blin_ref, wres_ref, bres_ref, a_ref, out_ref):
    dv = dinv_ref[...]
    s = sp_ref[0] + sp_ref[1] + hn_ref[...]
    z = dv * s + b2_ref[...] + h_ref[...]
    h3 = jnp.where(z >= 0, z, 0.01 * z)
    t1 = _mm_t(h3, wlin_ref[...]) + blin_ref[...]
    t2 = _mm_t(xp_ref[...], wres_ref[...]) + bres_ref[...]
    av = a_ref[0, 0]
    out_ref[...] = (jnp.where(t1 >= 0, t1, av * t1)
                    + jnp.where(t2 >= 0, t2, av * t2))


_row_spec = pl.BlockSpec((BLK, F), lambda i: (i, 0))
_col_spec = pl.BlockSpec((BLK, 1), lambda i: (i, 0))
_part_spec = pl.BlockSpec((2, BLK, F), lambda i: (0, i, 0))
_degp_spec = pl.BlockSpec((2, BLK, 1), lambda i: (0, i, 0))
_w_spec = pl.BlockSpec((F, F), lambda i: (0, 0))
_b_spec = pl.BlockSpec((1, F), lambda i: (0, 0))
_a_spec = pl.BlockSpec((1, 1), lambda i: (0, 0))

_rows_out = jax.ShapeDtypeStruct((NP, F), jnp.float32)
_col_out = jax.ShapeDtypeStruct((NP, 1), jnp.float32)

_prep_call = pl.pallas_call(
    _prep_body, grid=(GRID,),
    in_specs=[_row_spec, _degp_spec, _w_spec, _b_spec, _w_spec],
    out_specs=[_row_spec, _row_spec, _col_spec],
    out_shape=[_rows_out, _rows_out, _col_out],
)

_layer_call = pl.pallas_call(
    _layer_body, grid=(GRID,),
    in_specs=[_part_spec, _row_spec, _row_spec, _col_spec, _b_spec, _w_spec],
    out_specs=[_row_spec, _row_spec],
    out_shape=[_rows_out, _rows_out],
)

_final_call = pl.pallas_call(
    _final_body, grid=(GRID,),
    in_specs=[_part_spec, _row_spec, _row_spec, _col_spec, _b_spec, _row_spec,
              _w_spec, _b_spec, _w_spec, _b_spec, _a_spec],
    out_specs=_row_spec,
    out_shape=_rows_out,
)


def kernel(x, edge_index, Wproj, bproj, W0, b0, W1, b1, W2, b2,
           Wlin, blin, Wres, bres, a):
    f32 = jnp.float32
    pad_e = EP - E
    src2d = jnp.concatenate(
        [edge_index[0], jnp.full((pad_e,), N, jnp.int32)]).reshape(TOT_G, G)
    dst2d = jnp.concatenate(
        [edge_index[1], jnp.full((pad_e,), N, jnp.int32)]).reshape(TOT_G, G)
    x_pad = jnp.concatenate([x, jnp.zeros((NP - N, F), f32)])
    zeros1 = jnp.zeros((ROWS_PT,), f32)
    zrows = jnp.zeros((64, F), f32)

    degp = _deg_kernel(dst2d, zeros1).reshape(2, NP, 1)
    h, hn, dinv = _prep_call(x_pad, degp, Wproj, bproj.reshape(1, F), W0)

    for (b_l, w_next) in ((b0, W1), (b1, W2)):
        sp = _scatter_kernel(hn, src2d, dst2d, zrows)
        h, hn = _layer_call(sp, hn, h, dinv, b_l.reshape(1, F), w_next)

    sp = _scatter_kernel(hn, src2d, dst2d, zrows)
    out = _final_call(sp, hn, h, dinv, b2.reshape(1, F), x_pad,
                      Wlin, blin.reshape(1, F), Wres, bres.reshape(1, F),
                      a.reshape(1, 1))
    return out[:N]


# speedup vs baseline: 1.0959x; 1.0959x over previous
import functools

import jax
import jax.numpy as jnp
from jax import lax
from jax.experimental import pallas as pl
from jax.experimental.pallas import tpu as pltpu
from jax.experimental.pallas import tpu_sc as plsc

N = 10000
E = 320000
F = 128
NP = 10240
ROWS_PT = NP // 16
G = 64
CH = 16
N0G = 240
N1G = 80
TOT_G = 16 * (N0G + N1G)
EP = TOT_G * G
BLK = 1280
GRID = NP // BLK

_mesh = plsc.VectorSubcoreMesh(core_axis_name="c", subcore_axis_name="s")


@functools.partial(
    pl.kernel,
    out_type=jax.ShapeDtypeStruct((2, NP), jnp.float32),
    mesh=_mesh,
    scratch_types=[
        pltpu.VMEM((8, G), jnp.int32),
        pltpu.VMEM((G,), jnp.float32),
        pltpu.VMEM_SHARED((NP,), jnp.float32),
    ],
)
def _deg_kernel(dst_hbm, zeros1_hbm, degp_hbm, idx_v, ones_v, acc_sh):
    cid = lax.axis_index("c")
    tid = lax.axis_index("s")
    ng = TOT_G // 32
    base = (cid * 16 + tid) * ng
    pltpu.sync_copy(zeros1_hbm, acc_sh.at[pl.ds(tid * ROWS_PT, ROWS_PT)])
    for i in range(G // 16):
        ones_v[pl.ds(i * 16, 16)] = jnp.ones((16,), jnp.float32)
    plsc.subcore_barrier()

    def body(c, carry):
        pltpu.sync_copy(dst_hbm.at[pl.ds(base + c * 8, 8)], idx_v)

        def inner(k, carry2):
            pltpu.sync_copy(ones_v, acc_sh.at[idx_v.at[k]], add=True)
            return carry2

        return lax.fori_loop(0, 8, inner, carry)

    lax.fori_loop(0, ng // 8, body, 0)
    plsc.subcore_barrier()
    pltpu.sync_copy(acc_sh.at[pl.ds(tid * ROWS_PT, ROWS_PT)],
                    degp_hbm.at[cid, pl.ds(tid * ROWS_PT, ROWS_PT)])


@functools.partial(
    pl.kernel,
    out_type=jax.ShapeDtypeStruct((2, NP, F), jnp.float32),
    mesh=_mesh,
    scratch_types=[
        pltpu.VMEM((2, CH, G), jnp.int32),
        pltpu.VMEM((2, CH, G), jnp.int32),
        pltpu.VMEM((G, F), jnp.float32),
        pltpu.VMEM((G, F), jnp.float32),
        pltpu.VMEM((64, F), jnp.float32),
        pltpu.VMEM_SHARED((NP, F), jnp.float32),
        pltpu.SemaphoreType.DMA,
        pltpu.SemaphoreType.DMA,
        pltpu.SemaphoreType.DMA,
        pltpu.SemaphoreType.DMA,
    ],
)
def _scatter_kernel(hn_hbm, src_hbm, dst_hbm, zrows_hbm, out_hbm,
                    isrc, idst, buf_a, buf_b, zblk, acc_sh,
                    sem_a, sem_b, sem_si, sem_di):
    cid = lax.axis_index("c")
    tid = lax.axis_index("s")
    ng = jnp.where(cid == 0, N0G, N1G)
    nch = ng // CH
    base = jnp.where(cid == 0, tid * N0G, 16 * N0G + tid * N1G)
    @pl.when(cid == 0)
    def _():
        pltpu.sync_copy(zrows_hbm, acc_sh.at[pl.ds(tid * ROWS_PT, ROWS_PT)])

    @pl.when(cid == 1)
    def _():
        pltpu.sync_copy(zrows_hbm.at[pl.ds(0, 64)], zblk)
        for k in range(ROWS_PT // 64):
            pltpu.sync_copy(zblk, acc_sh.at[pl.ds(tid * ROWS_PT + k * 64, 64)])

    pltpu.sync_copy(src_hbm.at[pl.ds(base, CH)], isrc.at[0])
    pltpu.sync_copy(dst_hbm.at[pl.ds(base, CH)], idst.at[0])
    plsc.subcore_barrier()

    pltpu.async_copy(src_hbm.at[pl.ds(base + CH, CH)], isrc.at[1], sem_si)
    pltpu.async_copy(dst_hbm.at[pl.ds(base + CH, CH)], idst.at[1], sem_di)
    pltpu.async_copy(hn_hbm.at[isrc.at[0, 0]], buf_a, sem_a)
    pltpu.async_copy(hn_hbm.at[isrc.at[0, 1]], buf_b, sem_b)

    def body(g, carry):
        j = 2 * g
        c = j // CH
        r = j - c * CH

        @pl.when((r == CH - 2) & (c + 1 < nch))
        def _():
            pltpu.make_async_copy(src_hbm.at[pl.ds(base, CH)],
                                  isrc.at[0], sem_si).wait()
            pltpu.make_async_copy(dst_hbm.at[pl.ds(base, CH)],
                                  idst.at[0], sem_di).wait()

        def slotrow(jj):
            cc = jj // CH
            return lax.rem(cc, 2), jj - cc * CH

        s0, r0 = lax.rem(c, 2), r
        pltpu.make_async_copy(hn_hbm.at[isrc.at[s0, r0]], buf_a, sem_a).wait()
        pltpu.sync_copy(buf_a, acc_sh.at[idst.at[s0, r0]], add=True)

        @pl.when(j + 2 < ng)
        def _():
            s2, r2 = slotrow(j + 2)
            pltpu.async_copy(hn_hbm.at[isrc.at[s2, r2]], buf_a, sem_a)

        s1, r1 = slotrow(j + 1)
        pltpu.make_async_copy(hn_hbm.at[isrc.at[s1, r1]], buf_b, sem_b).wait()
        pltpu.sync_copy(buf_b, acc_sh.at[idst.at[s1, r1]], add=True)

        @pl.when(j + 3 < ng)
        def _():
            s3, r3 = slotrow(j + 3)
            pltpu.async_copy(hn_hbm.at[isrc.at[s3, r3]], buf_b, sem_b)

        @pl.when((r == CH - 2) & (c + 2 < nch))
        def _():
            slot = lax.rem(c, 2)
            off = base + (c + 2) * CH
            pltpu.async_copy(src_hbm.at[pl.ds(off, CH)], isrc.at[slot], sem_si)
            pltpu.async_copy(dst_hbm.at[pl.ds(off, CH)], idst.at[slot], sem_di)

        return carry

    lax.fori_loop(0, ng // 2, body, 0)
    plsc.subcore_barrier()
    pltpu.sync_copy(acc_sh.at[pl.ds(tid * ROWS_PT, ROWS_PT)],
                    out_hbm.at[cid, pl.ds(tid * ROWS_PT, ROWS_PT)])


def _mm_t(x, w):
    return lax.dot_general(x, w, (((1,), (1,)), ((), ())),
                           preferred_element_type=jnp.float32)


def _prep_body(xp_ref, degp_ref, wproj_ref, bproj_ref, w0_ref,
               h_ref, hn0_ref, dinv_ref):
    d = degp_ref[0] + degp_ref[1] + 1.0
    dv = lax.rsqrt(d)
    h = _mm_t(xp_ref[...], wproj_ref[...]) + bproj_ref[...]
    h_ref[...] = h
    hn0_ref[...] = dv * _mm_t(h, w0_ref[...])
    dinv_ref[...] = dv


def _layer_body(sp_ref, hn_ref, h_ref, dinv_ref, b_ref, wn_ref,
                hout_ref, hnout_ref):
    dv = dinv_ref[...]
    s = sp_ref[0] + sp_ref[1] + hn_ref[...]
    z = dv * s + b_ref[...] + h_ref[...]
    h_new = jnp.where(z >= 0, z, 0.01 * z)
    hout_ref[...] = h_new
    hnout_ref[...] = dv * _mm_t(h_new, wn_ref[...])


def _final_body(sp_ref, hn_ref, h_ref, dinv_ref, b2_ref, xp_ref,
                wlin_ref, blin_ref, wres_ref, bres_ref, a_ref, out_ref):
    dv = dinv_ref[...]
    s = sp_ref[0] + sp_ref[1] + hn_ref[...]
    z = dv * s + b2_ref[...] + h_ref[...]
    h3 = jnp.where(z >= 0, z, 0.01 * z)
    t1 = _mm_t(h3, wlin_ref[...]) + blin_ref[...]
    t2 = _mm_t(xp_ref[...], wres_ref[...]) + bres_ref[...]
    av = a_ref[0, 0]
    out_ref[...] = (jnp.where(t1 >= 0, t1, av * t1)
                    + jnp.where(t2 >= 0, t2, av * t2))


_row_spec = pl.BlockSpec((BLK, F), lambda i: (i, 0))
_col_spec = pl.BlockSpec((BLK, 1), lambda i: (i, 0))
_part_spec = pl.BlockSpec((2, BLK, F), lambda i: (0, i, 0))
_degp_spec = pl.BlockSpec((2, BLK, 1), lambda i: (0, i, 0))
_w_spec = pl.BlockSpec((F, F), lambda i: (0, 0))
_b_spec = pl.BlockSpec((1, F), lambda i: (0, 0))
_a_spec = pl.BlockSpec((1, 1), lambda i: (0, 0))

_rows_out = jax.ShapeDtypeStruct((NP, F), jnp.float32)
_col_out = jax.ShapeDtypeStruct((NP, 1), jnp.float32)

_prep_call = pl.pallas_call(
    _prep_body, grid=(GRID,),
    in_specs=[_row_spec, _degp_spec, _w_spec, _b_spec, _w_spec],
    out_specs=[_row_spec, _row_spec, _col_spec],
    out_shape=[_rows_out, _rows_out, _col_out],
)

_layer_call = pl.pallas_call(
    _layer_body, grid=(GRID,),
    in_specs=[_part_spec, _row_spec, _row_spec, _col_spec, _b_spec, _w_spec],
    out_specs=[_row_spec, _row_spec],
    out_shape=[_rows_out, _rows_out],
)

_final_call = pl.pallas_call(
    _final_body, grid=(GRID,),
    in_specs=[_part_spec, _row_spec, _row_spec, _col_spec, _b_spec, _row_spec,
              _w_spec, _b_spec, _w_spec, _b_spec, _a_spec],
    out_specs=_row_spec,
    out_shape=_rows_out,
)


def kernel(x, edge_index, Wproj, bproj, W0, b0, W1, b1, W2, b2,
           Wlin, blin, Wres, bres, a):
    f32 = jnp.float32
    pad_e = EP - E
    src2d = jnp.concatenate(
        [edge_index[0], jnp.full((pad_e,), N, jnp.int32)]).reshape(TOT_G, G)
    dst2d = jnp.concatenate(
        [edge_index[1], jnp.full((pad_e,), N, jnp.int32)]).reshape(TOT_G, G)
    x_pad = jnp.concatenate([x, jnp.zeros((NP - N, F), f32)])
    zeros1 = jnp.zeros((ROWS_PT,), f32)
    zrows = jnp.zeros((ROWS_PT, F), f32)

    degp = _deg_kernel(dst2d, zeros1).reshape(2, NP, 1)
    h, hn, dinv = _prep_call(x_pad, degp, Wproj, bproj.reshape(1, F), W0)

    for (b_l, w_next) in ((b0, W1), (b1, W2)):
        sp = _scatter_kernel(hn, src2d, dst2d, zrows)
        h, hn = _layer_call(sp, hn, h, dinv, b_l.reshape(1, F), w_next)

    sp = _scatter_kernel(hn, src2d, dst2d, zrows)
    out = _final_call(sp, hn, h, dinv, b2.reshape(1, F), x_pad,
                      Wlin, blin.reshape(1, F), Wres, bres.reshape(1, F),
                      a.reshape(1, 1))
    return out[:N]


# speedup vs baseline: 1.2845x; 1.1721x over previous
import functools

import jax
import jax.numpy as jnp
from jax import lax
from jax.experimental import pallas as pl
from jax.experimental.pallas import tpu as pltpu
from jax.experimental.pallas import tpu_sc as plsc

N = 10000
E = 320000
F = 128
NP = 10240
ROWS_PT = NP // 16
G = 64
CH = 16
N0G = 304
N1G = 16
TOT_G = 16 * (N0G + N1G)
EP = TOT_G * G
BLK = 1280
GRID = NP // BLK

_mesh = plsc.VectorSubcoreMesh(core_axis_name="c", subcore_axis_name="s")


@functools.partial(
    pl.kernel,
    out_type=jax.ShapeDtypeStruct((2, NP), jnp.float32),
    mesh=_mesh,
    scratch_types=[
        pltpu.VMEM((8, G), jnp.int32),
        pltpu.VMEM((G,), jnp.float32),
        pltpu.VMEM_SHARED((NP,), jnp.float32),
    ],
)
def _deg_kernel(dst_hbm, zeros1_hbm, degp_hbm, idx_v, ones_v, acc_sh):
    cid = lax.axis_index("c")
    tid = lax.axis_index("s")
    ng = TOT_G // 32
    base = (cid * 16 + tid) * ng
    pltpu.sync_copy(zeros1_hbm, acc_sh.at[pl.ds(tid * ROWS_PT, ROWS_PT)])
    for i in range(G // 16):
        ones_v[pl.ds(i * 16, 16)] = jnp.ones((16,), jnp.float32)
    plsc.subcore_barrier()

    def body(c, carry):
        pltpu.sync_copy(dst_hbm.at[pl.ds(base + c * 8, 8)], idx_v)

        def inner(k, carry2):
            pltpu.sync_copy(ones_v, acc_sh.at[idx_v.at[k]], add=True)
            return carry2

        return lax.fori_loop(0, 8, inner, carry)

    lax.fori_loop(0, ng // 8, body, 0)
    plsc.subcore_barrier()
    pltpu.sync_copy(acc_sh.at[pl.ds(tid * ROWS_PT, ROWS_PT)],
                    degp_hbm.at[cid, pl.ds(tid * ROWS_PT, ROWS_PT)])


@functools.partial(
    pl.kernel,
    out_type=jax.ShapeDtypeStruct((2, NP, F), jnp.float32),
    mesh=_mesh,
    scratch_types=[
        pltpu.VMEM((2, CH, G), jnp.int32),
        pltpu.VMEM((2, CH, G), jnp.int32),
        pltpu.VMEM((G, F), jnp.float32),
        pltpu.VMEM((G, F), jnp.float32),
        pltpu.VMEM((64, F), jnp.float32),
        pltpu.VMEM_SHARED((NP, F), jnp.float32),
        pltpu.SemaphoreType.DMA,
        pltpu.SemaphoreType.DMA,
        pltpu.SemaphoreType.DMA,
        pltpu.SemaphoreType.DMA,
    ],
)
def _scatter_kernel(hn_hbm, src_hbm, dst_hbm, zrows_hbm, out_hbm,
                    isrc, idst, buf_a, buf_b, zblk, acc_sh,
                    sem_a, sem_b, sem_si, sem_di):
    cid = lax.axis_index("c")
    tid = lax.axis_index("s")
    ng = jnp.where(cid == 0, N0G, N1G)
    nch = ng // CH
    base = jnp.where(cid == 0, tid * N0G, 16 * N0G + tid * N1G)
    @pl.when(cid == 0)
    def _():
        pltpu.sync_copy(zrows_hbm, acc_sh.at[pl.ds(tid * ROWS_PT, ROWS_PT)])

    @pl.when(cid == 1)
    def _():
        pltpu.sync_copy(zrows_hbm.at[pl.ds(0, 64)], zblk)
        for k in range(ROWS_PT // 64):
            pltpu.sync_copy(zblk, acc_sh.at[pl.ds(tid * ROWS_PT + k * 64, 64)])

    pltpu.sync_copy(src_hbm.at[pl.ds(base, CH)], isrc.at[0])
    pltpu.sync_copy(dst_hbm.at[pl.ds(base, CH)], idst.at[0])
    plsc.subcore_barrier()

    pltpu.async_copy(src_hbm.at[pl.ds(base + CH, CH)], isrc.at[1], sem_si)
    pltpu.async_copy(dst_hbm.at[pl.ds(base + CH, CH)], idst.at[1], sem_di)
    pltpu.async_copy(hn_hbm.at[isrc.at[0, 0]], buf_a, sem_a)
    pltpu.async_copy(hn_hbm.at[isrc.at[0, 1]], buf_b, sem_b)

    def body(g, carry):
        j = 2 * g
        c = j // CH
        r = j - c * CH

        @pl.when((r == CH - 2) & (c + 1 < nch))
        def _():
            pltpu.make_async_copy(src_hbm.at[pl.ds(base, CH)],
                                  isrc.at[0], sem_si).wait()
            pltpu.make_async_copy(dst_hbm.at[pl.ds(base, CH)],
                                  idst.at[0], sem_di).wait()

        def slotrow(jj):
            cc = jj // CH
            return lax.rem(cc, 2), jj - cc * CH

        s0, r0 = lax.rem(c, 2), r
        pltpu.make_async_copy(hn_hbm.at[isrc.at[s0, r0]], buf_a, sem_a).wait()
        pltpu.sync_copy(buf_a, acc_sh.at[idst.at[s0, r0]], add=True)

        @pl.when(j + 2 < ng)
        def _():
            s2, r2 = slotrow(j + 2)
            pltpu.async_copy(hn_hbm.at[isrc.at[s2, r2]], buf_a, sem_a)

        s1, r1 = slotrow(j + 1)
        pltpu.make_async_copy(hn_hbm.at[isrc.at[s1, r1]], buf_b, sem_b).wait()
        pltpu.sync_copy(buf_b, acc_sh.at[idst.at[s1, r1]], add=True)

        @pl.when(j + 3 < ng)
        def _():
            s3, r3 = slotrow(j + 3)
            pltpu.async_copy(hn_hbm.at[isrc.at[s3, r3]], buf_b, sem_b)

        @pl.when((r == CH - 2) & (c + 2 < nch))
        def _():
            slot = lax.rem(c, 2)
            off = base + (c + 2) * CH
            pltpu.async_copy(src_hbm.at[pl.ds(off, CH)], isrc.at[slot], sem_si)
            pltpu.async_copy(dst_hbm.at[pl.ds(off, CH)], idst.at[slot], sem_di)

        return carry

    lax.fori_loop(0, ng // 2, body, 0)
    plsc.subcore_barrier()
    pltpu.sync_copy(acc_sh.at[pl.ds(tid * ROWS_PT, ROWS_PT)],
                    out_hbm.at[cid, pl.ds(tid * ROWS_PT, ROWS_PT)])


def _mm_t(x, w):
    return lax.dot_general(x, w, (((1,), (1,)), ((), ())),
                           preferred_element_type=jnp.float32)


def _prep_body(xp_ref, degp_ref, wproj_ref, bproj_ref, w0_ref,
               h_ref, hn0_ref, dinv_ref):
    d = degp_ref[0] + degp_ref[1] + 1.0
    dv = lax.rsqrt(d)
    h = _mm_t(xp_ref[...], wproj_ref[...]) + bproj_ref[...]
    h_ref[...] = h
    hn0_ref[...] = dv * _mm_t(h, w0_ref[...])
    dinv_ref[...] = dv


def _layer_body(sp_ref, hn_ref, h_ref, dinv_ref, b_ref, wn_ref,
                hout_ref, hnout_ref):
    dv = dinv_ref[...]
    s = sp_ref[0] + sp_ref[1] + hn_ref[...]
    z = dv * s + b_ref[...] + h_ref[...]
    h_new = jnp.where(z >= 0, z, 0.01 * z)
    hout_ref[...] = h_new
    hnout_ref[...] = dv * _mm_t(h_new, wn_ref[...])


def _final_body(sp_ref, hn_ref, h_ref, dinv_ref, b2_ref, xp_ref,
                wlin_ref, blin_ref, wres_ref, bres_ref, a_ref, out_ref):
    dv = dinv_ref[...]
    s = sp_ref[0] + sp_ref[1] + hn_ref[...]
    z = dv * s + b2_ref[...] + h_ref[...]
    h3 = jnp.where(z >= 0, z, 0.01 * z)
    t1 = _mm_t(h3, wlin_ref[...]) + blin_ref[...]
    t2 = _mm_t(xp_ref[...], wres_ref[...]) + bres_ref[...]
    av = a_ref[0, 0]
    out_ref[...] = (jnp.where(t1 >= 0, t1, av * t1)
                    + jnp.where(t2 >= 0, t2, av * t2))


_row_spec = pl.BlockSpec((BLK, F), lambda i: (i, 0))
_col_spec = pl.BlockSpec((BLK, 1), lambda i: (i, 0))
_part_spec = pl.BlockSpec((2, BLK, F), lambda i: (0, i, 0))
_degp_spec = pl.BlockSpec((2, BLK, 1), lambda i: (0, i, 0))
_w_spec = pl.BlockSpec((F, F), lambda i: (0, 0))
_b_spec = pl.BlockSpec((1, F), lambda i: (0, 0))
_a_spec = pl.BlockSpec((1, 1), lambda i: (0, 0))

_rows_out = jax.ShapeDtypeStruct((NP, F), jnp.float32)
_col_out = jax.ShapeDtypeStruct((NP, 1), jnp.float32)

_prep_call = pl.pallas_call(
    _prep_body, grid=(GRID,),
    in_specs=[_row_spec, _degp_spec, _w_spec, _b_spec, _w_spec],
    out_specs=[_row_spec, _row_spec, _col_spec],
    out_shape=[_rows_out, _rows_out, _col_out],
)

_layer_call = pl.pallas_call(
    _layer_body, grid=(GRID,),
    in_specs=[_part_spec, _row_spec, _row_spec, _col_spec, _b_spec, _w_spec],
    out_specs=[_row_spec, _row_spec],
    out_shape=[_rows_out, _rows_out],
)

_final_call = pl.pallas_call(
    _final_body, grid=(GRID,),
    in_specs=[_part_spec, _row_spec, _row_spec, _col_spec, _b_spec, _row_spec,
              _w_spec, _b_spec, _w_spec, _b_spec, _a_spec],
    out_specs=_row_spec,
    out_shape=_rows_out,
)


def kernel(x, edge_index, Wproj, bproj, W0, b0, W1, b1, W2, b2,
           Wlin, blin, Wres, bres, a):
    f32 = jnp.float32
    pad_e = EP - E
    src2d = jnp.concatenate(
        [edge_index[0], jnp.full((pad_e,), N, jnp.int32)]).reshape(TOT_G, G)
    dst2d = jnp.concatenate(
        [edge_index[1], jnp.full((pad_e,), N, jnp.int32)]).reshape(TOT_G, G)
    x_pad = jnp.concatenate([x, jnp.zeros((NP - N, F), f32)])
    zeros1 = jnp.zeros((ROWS_PT,), f32)
    zrows = jnp.zeros((ROWS_PT, F), f32)

    degp = _deg_kernel(dst2d, zeros1).reshape(2, NP, 1)
    h, hn, dinv = _prep_call(x_pad, degp, Wproj, bproj.reshape(1, F), W0)

    for (b_l, w_next) in ((b0, W1), (b1, W2)):
        sp = _scatter_kernel(hn, src2d, dst2d, zrows)
        h, hn = _layer_call(sp, hn, h, dinv, b_l.reshape(1, F), w_next)

    sp = _scatter_kernel(hn, src2d, dst2d, zrows)
    out = _final_call(sp, hn, h, dinv, b2.reshape(1, F), x_pad,
                      Wlin, blin.reshape(1, F), Wres, bres.reshape(1, F),
                      a.reshape(1, 1))
    return out[:N]
